# baseline (speedup 1.0000x reference)
import functools

import jax
import jax.numpy as jnp
from jax import lax
from jax.experimental import pallas as pl
from jax.experimental.pallas import tpu as pltpu
from jax.experimental.pallas import tpu_sc as plsc

BATCH = 16384
EMBED = 64
NUM_CORES = 2
NUM_SUBCORES = 16
NUM_WORKERS = NUM_CORES * NUM_SUBCORES
ROWS_PER_W = BATCH // NUM_WORKERS
CHUNK = 128
NCHUNK = ROWS_PER_W // CHUNK
LANES = 16
GROUPS = CHUNK // LANES


def _body(uidx_hbm, iidx_hbm, utab_hbm, itab_hbm, out_hbm,
          uoix, ioix, ushf, ishf, urows, irows, outv, sem):
    wid = lax.axis_index("s") * NUM_CORES + lax.axis_index("c")
    base = wid * ROWS_PER_W

    for j in range(NCHUNK):
        pltpu.sync_copy(uidx_hbm.at[pl.ds(base + j * CHUNK, CHUNK)], uoix.at[j])
        pltpu.sync_copy(iidx_hbm.at[pl.ds(base + j * CHUNK, CHUNK)], ioix.at[j])
        for m in range(GROUPS):
            s = pl.ds(m * LANES, LANES)
            ushf[j, s] = lax.shift_right_logical(uoix[j, s], 1)
            ishf[j, s] = lax.shift_right_logical(ioix[j, s], 1)

        cu = pltpu.async_copy(utab_hbm.at[ushf.at[j]], urows, sem)
        ci = pltpu.async_copy(itab_hbm.at[ishf.at[j]], irows, sem)
        cu.wait()
        ci.wait()

        def group(g, _):
            s = pl.ds(g * LANES, LANES)
            cb_u = (uoix[j, s] & 1) * EMBED
            cb_i = (ioix[j, s] & 1) * EMBED
            rid = g * LANES + lax.iota(jnp.int32, LANES)
            acc = jnp.zeros((LANES,), jnp.float32)
            for k in range(EMBED):
                u = plsc.load_gather(urows, [rid, cb_u + k])
                v = plsc.load_gather(irows, [rid, cb_i + k])
                acc = acc + u * v
            outv[pl.ds(j * CHUNK + g * LANES, LANES)] = acc
            return 0

        lax.fori_loop(0, GROUPS, group, 0)

    pltpu.sync_copy(outv, out_hbm.at[pl.ds(base, ROWS_PER_W)])


def kernel(user_indices, item_indices, user_table, item_table):
    mesh = plsc.VectorSubcoreMesh(core_axis_name="c", subcore_axis_name="s")
    run = functools.partial(
        pl.kernel,
        out_type=jax.ShapeDtypeStruct((BATCH,), jnp.float32),
        mesh=mesh,
        compiler_params=pltpu.CompilerParams(needs_layout_passes=False),
        scratch_types=[
            pltpu.VMEM((NCHUNK, CHUNK), jnp.int32),
            pltpu.VMEM((NCHUNK, CHUNK), jnp.int32),
            pltpu.VMEM((NCHUNK, CHUNK), jnp.int32),
            pltpu.VMEM((NCHUNK, CHUNK), jnp.int32),
            pltpu.VMEM((CHUNK, 2 * EMBED), jnp.float32),
            pltpu.VMEM((CHUNK, 2 * EMBED), jnp.float32),
            pltpu.VMEM((ROWS_PER_W,), jnp.float32),
            pltpu.SemaphoreType.DMA,
        ],
    )(_body)
    ut = user_table.reshape(user_table.shape[0] // 2, 2 * EMBED)
    it = item_table.reshape(item_table.shape[0] // 2, 2 * EMBED)
    return run(user_indices.astype(jnp.int32), item_indices.astype(jnp.int32),
               ut, it)


# speedup vs baseline: 1.5405x; 1.5405x over previous
import functools

import jax
import jax.numpy as jnp
from jax import lax
from jax.experimental import pallas as pl
from jax.experimental.pallas import tpu as pltpu
from jax.experimental.pallas import tpu_sc as plsc

BATCH = 16384
EMBED = 64
NUM_CORES = 2
NUM_SUBCORES = 16
NUM_WORKERS = NUM_CORES * NUM_SUBCORES
ROWS_PER_W = BATCH // NUM_WORKERS
CHUNK = 128
NCHUNK = ROWS_PER_W // CHUNK
LANES = 16
GROUPS = CHUNK // LANES


def _body(uidx_hbm, iidx_hbm, utab_hbm, itab_hbm, out_hbm,
          uoix, ioix, urows, irows, outv, sem):
    wid = lax.axis_index("s") * NUM_CORES + lax.axis_index("c")
    base = wid * ROWS_PER_W

    def chunk_body(jc, _):
        cbase = base + jc * CHUNK
        pltpu.sync_copy(uidx_hbm.at[pl.ds(cbase, CHUNK)], uoix)
        pltpu.sync_copy(iidx_hbm.at[pl.ds(cbase, CHUNK)], ioix)

        copies = []
        for m in range(GROUPS):
            uvec = uoix[pl.ds(m * LANES, LANES)]
            ivec = ioix[pl.ds(m * LANES, LANES)]
            for l in range(LANES):
                r = m * LANES + l
                copies.append(pltpu.async_copy(
                    utab_hbm.at[uvec[l]], urows.at[r, pl.ds(0, EMBED)], sem))
                copies.append(pltpu.async_copy(
                    itab_hbm.at[ivec[l]], irows.at[r, pl.ds(0, EMBED)], sem))
        for c in copies:
            c.wait()

        def group(g, _):
            rid = g * LANES + lax.iota(jnp.int32, LANES)
            acc = jnp.zeros((LANES,), jnp.float32)
            for k in range(EMBED):
                ck = jnp.full((LANES,), k, jnp.int32)
                u = plsc.load_gather(urows, [rid, ck])
                v = plsc.load_gather(irows, [rid, ck])
                acc = acc + u * v
            outv[pl.ds(jc * CHUNK + g * LANES, LANES)] = acc
            return 0

        lax.fori_loop(0, GROUPS, group, 0)
        return 0

    lax.fori_loop(0, NCHUNK, chunk_body, 0)

    pltpu.sync_copy(outv, out_hbm.at[pl.ds(base, ROWS_PER_W)])


def kernel(user_indices, item_indices, user_table, item_table):
    mesh = plsc.VectorSubcoreMesh(core_axis_name="c", subcore_axis_name="s")
    run = functools.partial(
        pl.kernel,
        out_type=jax.ShapeDtypeStruct((BATCH,), jnp.float32),
        mesh=mesh,
        compiler_params=pltpu.CompilerParams(needs_layout_passes=False),
        scratch_types=[
            pltpu.VMEM((CHUNK,), jnp.int32),
            pltpu.VMEM((CHUNK,), jnp.int32),
            pltpu.VMEM((CHUNK, 2 * EMBED), jnp.float32),
            pltpu.VMEM((CHUNK, 2 * EMBED), jnp.float32),
            pltpu.VMEM((ROWS_PER_W,), jnp.float32),
            pltpu.SemaphoreType.DMA,
        ],
    )(_body)
    return run(user_indices.astype(jnp.int32), item_indices.astype(jnp.int32),
               user_table, item_table)
